# baseline (speedup 1.0000x reference)
import functools

import jax
import jax.numpy as jnp
from jax import lax
from jax.experimental import pallas as pl
from jax.experimental.pallas import tpu as pltpu
from jax.experimental.pallas import tpu_sc as plsc

_NC = 2
_NS = 16
_NW = _NC * _NS
_CH = 320
_NBUF = 4


def _sc_body(nch, d, idx_hbm, table_hbm, y_hbm, idx_v,
             r0, r1, r2, r3, gs0, gs1, gs2, gs3, ws0, ws1, ws2, ws3):
    rows = (r0, r1, r2, r3)
    gsem = (gs0, gs1, gs2, gs3)
    wsem = (ws0, ws1, ws2, ws3)
    wid = lax.axis_index("s") * _NC + lax.axis_index("c")
    pltpu.sync_copy(idx_hbm.at[wid], idx_v)
    base = wid * (nch * _CH)

    def issue_gather(c, b):
        pltpu.async_copy(table_hbm.at[idx_v.at[c]], rows[b], gsem[b])

    def issue_wb(c, b):
        pltpu.async_copy(
            rows[b], y_hbm.at[pl.ds(base + c * _CH, _CH), pl.ds(0, d)], wsem[b])

    def wait_g(b):
        pltpu.make_async_copy(table_hbm.at[idx_v.at[0]], rows[b], gsem[b]).wait()

    def wait_w(b):
        pltpu.make_async_copy(
            rows[b], y_hbm.at[pl.ds(base, _CH), pl.ds(0, d)], wsem[b]).wait()

    issue_gather(0, 0)
    issue_gather(1, 1)
    issue_gather(2, 2)
    wait_g(0)
    issue_wb(0, 0)
    issue_gather(3, 3)
    wait_g(1)
    issue_wb(1, 1)

    @pl.loop(2, nch - 2, step=4)
    def _(g):
        for k in range(4):
            j = g + k
            b = (2 + k) % 4
            bn = (b + 2) % 4
            wait_w(bn)
            issue_gather(j + 2, bn)
            wait_g(b)
            issue_wb(j, b)

    for j in (nch - 2, nch - 1):
        b = j % 4
        wait_g(b)
        issue_wb(j, b)
    for b in range(_NBUF):
        wait_w(b)


@functools.partial(jax.jit, static_argnums=(2, 3, 4))
def _impl(idx, table, b, nch, d):
    mesh = plsc.VectorSubcoreMesh(core_axis_name="c", subcore_axis_name="s")
    gather = pl.kernel(
        functools.partial(_sc_body, nch, d),
        out_type=jax.ShapeDtypeStruct((b, 2 * d), jnp.float32),
        mesh=mesh,
        scratch_types=(
            [pltpu.VMEM((nch, _CH), jnp.int32)]
            + [pltpu.VMEM((_CH, d), jnp.float32)] * _NBUF
            + [pltpu.SemaphoreType.DMA] * (2 * _NBUF)
        ),
        compiler_params=pltpu.CompilerParams(use_tc_tiling_on_sc=False),
    )
    return gather(idx, table)


def _fmt_body(y_ref, o_ref):
    o_ref[...] = y_ref[:, :, :64]


def kernel(x, table):
    s, l = x.shape
    v, d = table.shape
    b = s * l
    nch = b // (_NW * _CH)
    idx = x.astype(jnp.int32).reshape(_NW, nch, _CH)
    y = _impl(idx, table, b, nch, d)

    bs = 8
    fmt = pl.pallas_call(
        _fmt_body,
        grid=(s // bs,),
        in_specs=[pl.BlockSpec((bs, l, 2 * d), lambda i: (i, 0, 0))],
        out_specs=pl.BlockSpec((bs, l, d), lambda i: (i, 0, 0)),
        out_shape=jax.ShapeDtypeStruct((s, l, d), jnp.float32),
    )
    return fmt(y.reshape(s, l, 2 * d))
